# Initial kernel scaffold; baseline (speedup 1.0000x reference)
import jax
import jax.numpy as jnp
from jax.experimental import pallas as pl


def kernel(y, mu, low_rank, diag):
    raise NotImplementedError("write your pallas kernel here")


# speedup vs baseline: 1.2795x; 1.2795x over previous
import functools

import jax
import jax.numpy as jnp
from jax import lax
from jax.experimental import pallas as pl
from jax.experimental.pallas import tpu as pltpu
from jax.experimental.pallas import tpu_sc as plsc

_N = 100000
_Z = 64
_R = 16
_B = 4096

_NW = 32
_BPW = _B // _NW
_CH = 64


def _sc_gather(y, lr2, md):
    mesh = plsc.VectorSubcoreMesh(core_axis_name="c", subcore_axis_name="s")

    @functools.partial(
        pl.kernel,
        mesh=mesh,
        out_type=(
            jax.ShapeDtypeStruct((_B, _Z * _R), jnp.float32),
            jax.ShapeDtypeStruct((_B, 2 * _Z), jnp.float32),
        ),
        scratch_types=[
            pltpu.VMEM((_BPW,), jnp.int32),
            pltpu.VMEM((_CH, _Z * _R), jnp.float32),
            pltpu.VMEM((_BPW, 2 * _Z), jnp.float32),
            pltpu.SemaphoreType.DMA,
        ],
    )
    def k(y_hbm, lr_hbm, md_hbm, lrg_hbm, mdg_hbm, idx_v, rows_v, md_v, sem):
        wid = lax.axis_index("s") * 2 + lax.axis_index("c")
        base = wid * _BPW
        pltpu.sync_copy(y_hbm.at[pl.ds(base, _BPW)], idx_v)
        pltpu.async_copy(md_hbm.at[idx_v], md_v, sem).wait()
        pltpu.sync_copy(md_v, mdg_hbm.at[pl.ds(base, _BPW)])
        for c in range(_BPW // _CH):
            idx_c = idx_v.at[pl.ds(c * _CH, _CH)]
            pltpu.async_copy(lr_hbm.at[idx_c], rows_v, sem).wait()
            pltpu.sync_copy(rows_v, lrg_hbm.at[pl.ds(base + c * _CH, _CH)])

    return k(y, lr2, md)


_BC = 256


def _tc_body(lrg_ref, mdg_ref, out_ref):
    gt = lrg_ref[...].T
    mdt = mdg_ref[...].T
    mu_t = mdt[0:_Z]
    sp = jax.nn.softplus(mdt[_Z:2 * _Z])
    out_ref[0] = mu_t
    for j in range(_Z):
        if j > 0:
            out_ref[1 + j, 0:j] = jnp.zeros((j, _BC), jnp.float32)
        out_ref[1 + j, j:j + 1] = sp[j:j + 1]
        if j < _Z - 1:
            n = _Z - 1 - j
            acc = jnp.zeros((n, _BC), jnp.float32)
            for k in range(_R):
                a = lax.slice(gt, (_Z * k + j + 1, 0), (_Z * k + _Z, _BC))
                bj = lax.slice(gt, (_Z * k + j, 0), (_Z * k + j + 1, _BC))
                acc = acc + a * bj
            out_ref[1 + j, j + 1:_Z] = acc


def _tc_build(lrg, mdg):
    return pl.pallas_call(
        _tc_body,
        grid=(_B // _BC,),
        in_specs=[
            pl.BlockSpec((_BC, _Z * _R), lambda g: (g, 0)),
            pl.BlockSpec((_BC, 2 * _Z), lambda g: (g, 0)),
        ],
        out_specs=pl.BlockSpec((_Z + 1, _Z, _BC), lambda g: (0, 0, g)),
        out_shape=jax.ShapeDtypeStruct((_Z + 1, _Z, _B), jnp.float32),
    )(lrg, mdg)


def kernel(y, mu, low_rank, diag):
    lr2 = jnp.transpose(low_rank, (0, 2, 1)).reshape(_N, _Z * _R)
    md = jnp.concatenate([mu, diag], axis=1)
    lrg, mdg = _sc_gather(y, lr2, md)
    out_t = _tc_build(lrg, mdg)
    return jnp.transpose(out_t, (2, 1, 0))


# speedup vs baseline: 1.8763x; 1.4664x over previous
import functools

import jax
import jax.numpy as jnp
from jax import lax
from jax.experimental import pallas as pl
from jax.experimental.pallas import tpu as pltpu
from jax.experimental.pallas import tpu_sc as plsc

_N = 100000
_Z = 64
_R = 16
_B = 4096

_NW = 32
_BPW = _B // _NW
_CH = 64


def _sc_gather(y, lr2, md):
    mesh = plsc.VectorSubcoreMesh(core_axis_name="c", subcore_axis_name="s")

    @functools.partial(
        pl.kernel,
        mesh=mesh,
        out_type=(
            jax.ShapeDtypeStruct((_B, _Z * _R), jnp.float32),
            jax.ShapeDtypeStruct((_B, 2 * _Z), jnp.float32),
        ),
        scratch_types=[
            pltpu.VMEM((_BPW,), jnp.int32),
            pltpu.VMEM((_CH, _Z * _R), jnp.float32),
            pltpu.VMEM((_BPW, 2 * _Z), jnp.float32),
            pltpu.SemaphoreType.DMA,
        ],
    )
    def k(y_hbm, lr_hbm, md_hbm, lrg_hbm, mdg_hbm, idx_v, rows_v, md_v, sem):
        wid = lax.axis_index("s") * 2 + lax.axis_index("c")
        base = wid * _BPW
        pltpu.sync_copy(y_hbm.at[pl.ds(base, _BPW)], idx_v)
        pltpu.async_copy(md_hbm.at[idx_v], md_v, sem).wait()
        pltpu.sync_copy(md_v, mdg_hbm.at[pl.ds(base, _BPW)])
        for c in range(_BPW // _CH):
            idx_c = idx_v.at[pl.ds(c * _CH, _CH)]
            pltpu.async_copy(lr_hbm.at[idx_c], rows_v, sem).wait()
            pltpu.sync_copy(rows_v, lrg_hbm.at[pl.ds(base + c * _CH, _CH)])

    return k(y, lr2, md)


_BC = 256


def _tc_body(lrg_ref, mdg_ref, out_ref):
    gt = lrg_ref[...].T
    gt3 = gt.reshape(_Z, _R, _BC)
    mdt = mdg_ref[...].T
    mu_t = mdt[0:_Z]
    sp = jax.nn.softplus(mdt[_Z:2 * _Z])
    out_ref[0] = mu_t
    for j in range(_Z):
        if j > 0:
            out_ref[1 + j, 0:j] = jnp.zeros((j, _BC), jnp.float32)
        out_ref[1 + j, j:j + 1] = sp[j:j + 1]
        if j < _Z - 1:
            pj = gt3[j]
            prod = gt3[j + 1:] * pj[None]
            out_ref[1 + j, j + 1:_Z] = prod.sum(axis=1)


def _tc_build(lrg, mdg):
    return pl.pallas_call(
        _tc_body,
        grid=(_B // _BC,),
        in_specs=[
            pl.BlockSpec((_BC, _Z * _R), lambda g: (g, 0)),
            pl.BlockSpec((_BC, 2 * _Z), lambda g: (g, 0)),
        ],
        out_specs=pl.BlockSpec((_Z + 1, _Z, _BC), lambda g: (0, 0, g)),
        out_shape=jax.ShapeDtypeStruct((_Z + 1, _Z, _B), jnp.float32),
    )(lrg, mdg)


def kernel(y, mu, low_rank, diag):
    lr2 = low_rank.reshape(_N, _Z * _R)
    md = jnp.concatenate([mu, diag], axis=1)
    lrg, mdg = _sc_gather(y, lr2, md)
    out_t = _tc_build(lrg, mdg)
    return jnp.transpose(out_t, (2, 1, 0))
